# baseline (speedup 1.0000x reference)
import functools

import jax
import jax.numpy as jnp
from jax import lax
from jax.experimental import pallas as pl
from jax.experimental.pallas import tpu as pltpu
from jax.experimental.pallas import tpu_sc as plsc

B = 16384
L = 50
D = 32
VOCAB_ROWS = 1000000
NUM_CLASSES = 2

NC = 2
NS = 16
NW = NC * NS

SPW = B // NW
CS = 8
NCH = SPW // CS
IDX_PER_CHUNK = CS * L
GW = 50
NG = IDX_PER_CHUNK // GW
DP = 128


def _body(idx_hbm, table_hbm, wb_hbm, out_hbm,
          idx_a, idx_b, rows_a, rows_b, sums_v, out_v, wb_v, sem_a, sem_b):
    wid = lax.axis_index("c") * NS + lax.axis_index("s")

    def stage(ci, idx_v, rows_v, sem):
        pltpu.sync_copy(idx_hbm.at[wid * NCH + ci], idx_v)
        for j in range(NG):
            pltpu.async_copy(table_hbm.at[idx_v.at[j]],
                             rows_v.at[pl.ds(j * GW, GW)], sem)

    def drain(rows_v, sem):
        for j in range(NG):
            pltpu.make_async_copy(table_hbm.at[idx_a.at[0]],
                                  rows_v.at[pl.ds(j * GW, GW)], sem).wait()

    def accum(ci, rows_v):
        def sample_body(s, carry2):
            r0 = s * L
            a0 = rows_v[r0, pl.ds(0, 16)]
            a1 = rows_v[r0, pl.ds(16, 16)]
            for l in range(1, L):
                a0 = a0 + rows_v[r0 + l, pl.ds(0, 16)]
                a1 = a1 + rows_v[r0 + l, pl.ds(16, 16)]
            sums_v[ci * CS + s, pl.ds(0, 16)] = a0
            sums_v[ci * CS + s, pl.ds(16, 16)] = a1
            return carry2

        lax.fori_loop(0, CS, sample_body, 0)

    stage(0, idx_a, rows_a, sem_a)
    stage(1, idx_b, rows_b, sem_b)

    def pair_body(k, carry):
        ci = 2 * k

        drain(rows_a, sem_a)
        accum(ci, rows_a)

        @pl.when(ci + 2 < NCH)
        def _():
            stage(ci + 2, idx_a, rows_a, sem_a)

        drain(rows_b, sem_b)
        accum(ci + 1, rows_b)

        @pl.when(ci + 3 < NCH)
        def _():
            stage(ci + 3, idx_b, rows_b, sem_b)

        return carry

    lax.fori_loop(0, NCH // 2, pair_body, 0)

    pltpu.sync_copy(wb_hbm, wb_v)
    wvecs = [wb_v[pl.ds(k * 16, 16)] for k in range(5)]
    w0 = [wvecs[d // 16][d % 16] for d in range(D)]
    w1 = [wvecs[2 + d // 16][d % 16] for d in range(D)]
    b0 = wvecs[4][0]
    b1 = wvecs[4][1]
    lane = lax.iota(jnp.int32, 16)
    inv_l = jnp.float32(1.0 / L)

    def fc_body(g, carry):
        rows = g * 16 + lane
        l0 = jnp.zeros((16,), jnp.float32)
        l1 = jnp.zeros((16,), jnp.float32)
        for d in range(D):
            col = jnp.full((16,), d, jnp.int32)
            x = plsc.load_gather(sums_v, [rows, col])
            l0 = l0 + x * w0[d]
            l1 = l1 + x * w1[d]
        l0 = l0 * inv_l + b0
        l1 = l1 * inv_l + b1
        plsc.store_scatter(out_v, [rows * 2], l0)
        plsc.store_scatter(out_v, [rows * 2 + 1], l1)
        return carry

    lax.fori_loop(0, SPW // 16, fc_body, 0)
    pltpu.sync_copy(out_v, out_hbm.at[pl.ds(wid * SPW * NUM_CLASSES,
                                            SPW * NUM_CLASSES)])


@jax.jit
def _run(idx2d, table, wb):
    mesh = plsc.VectorSubcoreMesh(core_axis_name="c", subcore_axis_name="s")
    kfn = functools.partial(
        pl.kernel,
        mesh=mesh,
        compiler_params=pltpu.CompilerParams(
            needs_layout_passes=False, use_tc_tiling_on_sc=False),
        out_type=jax.ShapeDtypeStruct((B * NUM_CLASSES,), jnp.float32),
        scratch_types=[
            pltpu.VMEM((NG, GW), jnp.int32),
            pltpu.VMEM((NG, GW), jnp.int32),
            pltpu.VMEM((IDX_PER_CHUNK, DP), jnp.float32),
            pltpu.VMEM((IDX_PER_CHUNK, DP), jnp.float32),
            pltpu.VMEM((SPW, D), jnp.float32),
            pltpu.VMEM((SPW * NUM_CLASSES,), jnp.float32),
            pltpu.VMEM((80,), jnp.float32),
            pltpu.SemaphoreType.DMA,
            pltpu.SemaphoreType.DMA,
        ],
    )(_body)
    return kfn(idx2d, table, wb)


def kernel(input_ids, embed_table, fc_w, fc_b):
    embed_table = jnp.pad(embed_table, ((0, 0), (0, DP - D)))
    idx2d = input_ids.astype(jnp.int32).reshape(NW * NCH, NG, GW)
    b_pad = jnp.zeros((16,), jnp.float32).at[:NUM_CLASSES].set(fc_b)
    wb = jnp.concatenate([fc_w[:, 0], fc_w[:, 1], b_pad])
    out = _run(idx2d, embed_table, wb)
    return out.reshape(B, NUM_CLASSES)


# speedup vs baseline: 1.0193x; 1.0193x over previous
import functools

import jax
import jax.numpy as jnp
from jax import lax
from jax.experimental import pallas as pl
from jax.experimental.pallas import tpu as pltpu
from jax.experimental.pallas import tpu_sc as plsc

B = 16384
L = 50
D = 32
VOCAB_ROWS = 1000000
NUM_CLASSES = 2

NC = 2
NS = 16
NW = NC * NS

SPW = B // NW
CS = 32
NCH = SPW // CS
IDX_PER_CHUNK = CS * L
GW = 64
NG = IDX_PER_CHUNK // GW


def _body(idx_hbm, table_hbm, wb_hbm, out_hbm,
          idx_a, idx_b, rows_a, rows_b, sums_v, out_v, wb_v, sem_a, sem_b):
    wid = lax.axis_index("c") * NS + lax.axis_index("s")

    def stage(ci, idx_v, rows_v, sem):
        pltpu.sync_copy(idx_hbm.at[wid * NCH + ci], idx_v)
        for j in range(NG):
            pltpu.async_copy(table_hbm.at[idx_v.at[j]],
                             rows_v.at[pl.ds(j * GW, GW)], sem)

    def drain(rows_v, sem):
        for j in range(NG):
            pltpu.make_async_copy(table_hbm.at[idx_a.at[0]],
                                  rows_v.at[pl.ds(j * GW, GW)], sem).wait()

    def accum(ci, rows_v):
        def sample_body(s, carry2):
            r0 = s * L
            a0 = jnp.zeros((16,), jnp.float32)
            a1 = jnp.zeros((16,), jnp.float32)
            for l in range(L):
                row = rows_v[r0 + l, pl.ds(0, 32)]
                e, o = plsc.unpack(row, format=plsc.PackFormat.INTERLEAVED)
                a0 = a0 + e
                a1 = a1 + o
            sums_v[ci * CS + s, pl.ds(0, 16)] = a0
            sums_v[ci * CS + s, pl.ds(16, 16)] = a1
            return carry2

        lax.fori_loop(0, CS, sample_body, 0)

    stage(0, idx_a, rows_a, sem_a)
    stage(1, idx_b, rows_b, sem_b)

    def pair_body(k, carry):
        ci = 2 * k

        drain(rows_a, sem_a)
        accum(ci, rows_a)

        @pl.when(ci + 2 < NCH)
        def _():
            stage(ci + 2, idx_a, rows_a, sem_a)

        drain(rows_b, sem_b)
        accum(ci + 1, rows_b)

        @pl.when(ci + 3 < NCH)
        def _():
            stage(ci + 3, idx_b, rows_b, sem_b)

        return carry

    lax.fori_loop(0, NCH // 2, pair_body, 0)

    pltpu.sync_copy(wb_hbm, wb_v)
    wvecs = [wb_v[pl.ds(k * 16, 16)] for k in range(5)]
    dim_of_col = [2 * p if p < 16 else 2 * (p - 16) + 1 for p in range(D)]
    w0 = [wvecs[dim_of_col[p] // 16][dim_of_col[p] % 16] for p in range(D)]
    w1 = [wvecs[2 + dim_of_col[p] // 16][dim_of_col[p] % 16] for p in range(D)]
    b0 = wvecs[4][0]
    b1 = wvecs[4][1]
    lane = lax.iota(jnp.int32, 16)
    inv_l = jnp.float32(1.0 / L)

    def fc_body(g, carry):
        rows = g * 16 + lane
        l0 = jnp.zeros((16,), jnp.float32)
        l1 = jnp.zeros((16,), jnp.float32)
        for d in range(D):
            col = jnp.full((16,), d, jnp.int32)
            x = plsc.load_gather(sums_v, [rows, col])
            l0 = l0 + x * w0[d]
            l1 = l1 + x * w1[d]
        l0 = l0 * inv_l + b0
        l1 = l1 * inv_l + b1
        plsc.store_scatter(out_v, [rows * 2], l0)
        plsc.store_scatter(out_v, [rows * 2 + 1], l1)
        return carry

    lax.fori_loop(0, SPW // 16, fc_body, 0)
    pltpu.sync_copy(out_v, out_hbm.at[pl.ds(wid * SPW * NUM_CLASSES,
                                            SPW * NUM_CLASSES)])


@jax.jit
def _run(idx2d, table, wb):
    mesh = plsc.VectorSubcoreMesh(core_axis_name="c", subcore_axis_name="s")
    kfn = functools.partial(
        pl.kernel,
        mesh=mesh,
        compiler_params=pltpu.CompilerParams(
            needs_layout_passes=False, use_tc_tiling_on_sc=False),
        out_type=jax.ShapeDtypeStruct((B * NUM_CLASSES,), jnp.float32),
        scratch_types=[
            pltpu.VMEM((NG, GW), jnp.int32),
            pltpu.VMEM((NG, GW), jnp.int32),
            pltpu.VMEM((IDX_PER_CHUNK, D), jnp.bfloat16),
            pltpu.VMEM((IDX_PER_CHUNK, D), jnp.bfloat16),
            pltpu.VMEM((SPW, D), jnp.float32),
            pltpu.VMEM((SPW * NUM_CLASSES,), jnp.float32),
            pltpu.VMEM((80,), jnp.float32),
            pltpu.SemaphoreType.DMA,
            pltpu.SemaphoreType.DMA,
        ],
    )(_body)
    return kfn(idx2d, table, wb)


def kernel(input_ids, embed_table, fc_w, fc_b):
    embed_table = embed_table.astype(jnp.bfloat16)
    idx2d = input_ids.astype(jnp.int32).reshape(NW * NCH, NG, GW)
    b_pad = jnp.zeros((16,), jnp.float32).at[:NUM_CLASSES].set(fc_b)
    wb = jnp.concatenate([fc_w[:, 0], fc_w[:, 1], b_pad])
    out = _run(idx2d, embed_table, wb)
    return out.reshape(B, NUM_CLASSES)


# speedup vs baseline: 1.1947x; 1.1720x over previous
import functools

import jax
import jax.numpy as jnp
from jax import lax
from jax.experimental import pallas as pl
from jax.experimental.pallas import tpu as pltpu
from jax.experimental.pallas import tpu_sc as plsc

B = 16384
L = 50
D = 32
VOCAB_ROWS = 1000000
NUM_CLASSES = 2

NC = 2
NS = 16
NW = NC * NS

SPW = B // NW
CS = 32
NCH = SPW // CS
IDX_PER_CHUNK = CS * L
GW = 100
NG = IDX_PER_CHUNK // GW


def _body(idx_hbm, table_hbm, wb_hbm, out_hbm,
          idx_a, idx_b, rows_a, rows_b, sums_v, out_v, wb_v, sem_a, sem_b):
    wid = lax.axis_index("c") * NS + lax.axis_index("s")

    def stage(ci, idx_v, rows_v, sem):
        pltpu.sync_copy(idx_hbm.at[wid * NCH + ci], idx_v)
        for j in range(NG):
            pltpu.async_copy(table_hbm.at[idx_v.at[j]],
                             rows_v.at[pl.ds(j * GW, GW)], sem)

    def drain(rows_v, sem):
        for j in range(NG):
            pltpu.make_async_copy(table_hbm.at[idx_a.at[0]],
                                  rows_v.at[pl.ds(j * GW, GW)], sem).wait()

    def accum(ci, rows_v):
        def sample_body(s, carry2):
            r0 = s * L
            a0 = rows_v[r0, pl.ds(0, 16)]
            a1 = rows_v[r0, pl.ds(16, 16)]
            for l in range(1, L):
                a0 = a0 + rows_v[r0 + l, pl.ds(0, 16)]
                a1 = a1 + rows_v[r0 + l, pl.ds(16, 16)]
            sums_v[ci * CS + s, pl.ds(0, 16)] = a0
            sums_v[ci * CS + s, pl.ds(16, 16)] = a1
            return carry2

        lax.fori_loop(0, CS, sample_body, 0)

    stage(0, idx_a, rows_a, sem_a)
    stage(1, idx_b, rows_b, sem_b)

    def pair_body(k, carry):
        ci = 2 * k

        drain(rows_a, sem_a)
        accum(ci, rows_a)

        @pl.when(ci + 2 < NCH)
        def _():
            stage(ci + 2, idx_a, rows_a, sem_a)

        drain(rows_b, sem_b)
        accum(ci + 1, rows_b)

        @pl.when(ci + 3 < NCH)
        def _():
            stage(ci + 3, idx_b, rows_b, sem_b)

        return carry

    lax.fori_loop(0, NCH // 2, pair_body, 0)

    pltpu.sync_copy(wb_hbm, wb_v)
    wvecs = [wb_v[pl.ds(k * 16, 16)] for k in range(5)]
    w0 = [wvecs[d // 16][d % 16] for d in range(D)]
    w1 = [wvecs[2 + d // 16][d % 16] for d in range(D)]
    b0 = wvecs[4][0]
    b1 = wvecs[4][1]
    lane = lax.iota(jnp.int32, 16)
    inv_l = jnp.float32(1.0 / L)

    def fc_body(g, carry):
        rows = g * 16 + lane
        l0 = jnp.zeros((16,), jnp.float32)
        l1 = jnp.zeros((16,), jnp.float32)
        for d in range(D):
            col = jnp.full((16,), d, jnp.int32)
            x = plsc.load_gather(sums_v, [rows, col])
            l0 = l0 + x * w0[d]
            l1 = l1 + x * w1[d]
        l0 = l0 * inv_l + b0
        l1 = l1 * inv_l + b1
        plsc.store_scatter(out_v, [rows * 2], l0)
        plsc.store_scatter(out_v, [rows * 2 + 1], l1)
        return carry

    lax.fori_loop(0, SPW // 16, fc_body, 0)
    pltpu.sync_copy(out_v, out_hbm.at[pl.ds(wid * SPW * NUM_CLASSES,
                                            SPW * NUM_CLASSES)])


@jax.jit
def _run(idx2d, table, wb):
    mesh = plsc.VectorSubcoreMesh(core_axis_name="c", subcore_axis_name="s")
    kfn = functools.partial(
        pl.kernel,
        mesh=mesh,
        compiler_params=pltpu.CompilerParams(
            needs_layout_passes=False, use_tc_tiling_on_sc=False),
        out_type=jax.ShapeDtypeStruct((B * NUM_CLASSES,), jnp.float32),
        scratch_types=[
            pltpu.VMEM((NG, GW), jnp.int32),
            pltpu.VMEM((NG, GW), jnp.int32),
            pltpu.VMEM((IDX_PER_CHUNK, D), jnp.float32),
            pltpu.VMEM((IDX_PER_CHUNK, D), jnp.float32),
            pltpu.VMEM((SPW, D), jnp.float32),
            pltpu.VMEM((SPW * NUM_CLASSES,), jnp.float32),
            pltpu.VMEM((80,), jnp.float32),
            pltpu.SemaphoreType.DMA,
            pltpu.SemaphoreType.DMA,
        ],
    )(_body)
    return kfn(idx2d, table, wb)


def kernel(input_ids, embed_table, fc_w, fc_b):
    idx2d = input_ids.astype(jnp.int32).reshape(NW * NCH, NG, GW)
    b_pad = jnp.zeros((16,), jnp.float32).at[:NUM_CLASSES].set(fc_b)
    wb = jnp.concatenate([fc_w[:, 0], fc_w[:, 1], b_pad])
    out = _run(idx2d, embed_table, wb)
    return out.reshape(B, NUM_CLASSES)


# speedup vs baseline: 1.1998x; 1.0043x over previous
import functools

import jax
import jax.numpy as jnp
from jax import lax
from jax.experimental import pallas as pl
from jax.experimental.pallas import tpu as pltpu
from jax.experimental.pallas import tpu_sc as plsc

B = 16384
L = 50
D = 32
VOCAB_ROWS = 1000000
NUM_CLASSES = 2

NC = 2
NS = 16
NW = NC * NS

SPW = B // NW
CS = 32
NCH = SPW // CS
IDX_PER_CHUNK = CS * L
GW = 64
NG = IDX_PER_CHUNK // GW


def _body(idx_hbm, table_hbm, wb_hbm, out_hbm,
          idx_a, idx_b, rows_a, rows_b, sums_v, out_v, wb_v, sem_a, sem_b):
    wid = lax.axis_index("c") * NS + lax.axis_index("s")

    def stage(ci, idx_v, rows_v, sem):
        pltpu.sync_copy(idx_hbm.at[wid * NCH + ci], idx_v)
        for j in range(NG):
            pltpu.async_copy(table_hbm.at[idx_v.at[j]],
                             rows_v.at[pl.ds(j * GW, GW)], sem)

    def drain(rows_v, sem):
        for j in range(NG):
            pltpu.make_async_copy(table_hbm.at[idx_a.at[0]],
                                  rows_v.at[pl.ds(j * GW, GW)], sem).wait()

    def accum(ci, rows_v):
        def sample_body(s, carry2):
            r0 = s * L
            a0 = rows_v[r0, pl.ds(0, 16)]
            a1 = rows_v[r0, pl.ds(16, 16)]
            for l in range(1, L):
                a0 = a0 + rows_v[r0 + l, pl.ds(0, 16)]
                a1 = a1 + rows_v[r0 + l, pl.ds(16, 16)]
            sums_v[ci * CS + s, pl.ds(0, 16)] = a0
            sums_v[ci * CS + s, pl.ds(16, 16)] = a1
            return carry2

        lax.fori_loop(0, CS, sample_body, 0)

    stage(0, idx_a, rows_a, sem_a)
    stage(1, idx_b, rows_b, sem_b)

    def pair_body(k, carry):
        ci = 2 * k

        drain(rows_a, sem_a)
        accum(ci, rows_a)

        @pl.when(ci + 2 < NCH)
        def _():
            stage(ci + 2, idx_a, rows_a, sem_a)

        drain(rows_b, sem_b)
        accum(ci + 1, rows_b)

        @pl.when(ci + 3 < NCH)
        def _():
            stage(ci + 3, idx_b, rows_b, sem_b)

        return carry

    lax.fori_loop(0, NCH // 2, pair_body, 0)

    pltpu.sync_copy(wb_hbm, wb_v)
    wvecs = [wb_v[pl.ds(k * 16, 16)] for k in range(5)]
    w0 = [wvecs[d // 16][d % 16] for d in range(D)]
    w1 = [wvecs[2 + d // 16][d % 16] for d in range(D)]
    b0 = wvecs[4][0]
    b1 = wvecs[4][1]
    lane = lax.iota(jnp.int32, 16)
    inv_l = jnp.float32(1.0 / L)

    def fc_body(g, carry):
        rows = g * 16 + lane
        l0 = jnp.zeros((16,), jnp.float32)
        l1 = jnp.zeros((16,), jnp.float32)
        for d in range(D):
            col = jnp.full((16,), d, jnp.int32)
            x = plsc.load_gather(sums_v, [rows, col])
            l0 = l0 + x * w0[d]
            l1 = l1 + x * w1[d]
        l0 = l0 * inv_l + b0
        l1 = l1 * inv_l + b1
        plsc.store_scatter(out_v, [rows * 2], l0)
        plsc.store_scatter(out_v, [rows * 2 + 1], l1)
        return carry

    lax.fori_loop(0, SPW // 16, fc_body, 0)
    pltpu.sync_copy(out_v, out_hbm.at[pl.ds(wid * SPW * NUM_CLASSES,
                                            SPW * NUM_CLASSES)])


@jax.jit
def _run(idx2d, table, wb):
    mesh = plsc.VectorSubcoreMesh(core_axis_name="c", subcore_axis_name="s")
    kfn = functools.partial(
        pl.kernel,
        mesh=mesh,
        compiler_params=pltpu.CompilerParams(
            needs_layout_passes=False, use_tc_tiling_on_sc=False),
        out_type=jax.ShapeDtypeStruct((B * NUM_CLASSES,), jnp.float32),
        scratch_types=[
            pltpu.VMEM((NG, GW), jnp.int32),
            pltpu.VMEM((NG, GW), jnp.int32),
            pltpu.VMEM((IDX_PER_CHUNK, D), jnp.float32),
            pltpu.VMEM((IDX_PER_CHUNK, D), jnp.float32),
            pltpu.VMEM((SPW, D), jnp.float32),
            pltpu.VMEM((SPW * NUM_CLASSES,), jnp.float32),
            pltpu.VMEM((80,), jnp.float32),
            pltpu.SemaphoreType.DMA,
            pltpu.SemaphoreType.DMA,
        ],
    )(_body)
    return kfn(idx2d, table, wb)


def kernel(input_ids, embed_table, fc_w, fc_b):
    idx2d = input_ids.astype(jnp.int32).reshape(NW * NCH, NG, GW)
    b_pad = jnp.zeros((16,), jnp.float32).at[:NUM_CLASSES].set(fc_b)
    wb = jnp.concatenate([fc_w[:, 0], fc_w[:, 1], b_pad])
    out = _run(idx2d, embed_table, wb)
    return out.reshape(B, NUM_CLASSES)
